# Initial kernel scaffold; baseline (speedup 1.0000x reference)
import jax
import jax.numpy as jnp
from jax.experimental import pallas as pl


def kernel(testId, assessmentItemID, KnowledgeTag, interaction, question_N, W_test, W_question, W_tag, W_inter, W_qN, W_proj, b_proj, ln_g, ln_b):
    raise NotImplementedError("write your pallas kernel here")


# speedup vs baseline: 13.8578x; 13.8578x over previous
import functools

import jax
import jax.numpy as jnp
from jax import lax
from jax.experimental import pallas as pl
from jax.experimental.pallas import tpu as pltpu
from jax.experimental.pallas import tpu_sc as plsc

_B, _L = 4096, 200
_HD = 64
_INTD = 32
_NT = _B * _L
_NW = 32
_TPW = _NT // _NW
_TB = 512
_NB = _TPW // _TB
_NC = _TB // 128


def _proj_pair_body(s0, s1, u_ref, v_ref, wproj, t_ref):
    f32 = jnp.float32
    p_u = wproj[s0:s0 + _INTD, :]
    p_v = wproj[s1:s1 + _INTD, :]
    t_ref[...] = (jnp.dot(u_ref[...], p_u, preferred_element_type=f32)
                  + jnp.dot(v_ref[...], p_v, preferred_element_type=f32))


def _proj_test_body(u_ref, wproj, bproj, t_ref):
    t_ref[...] = (jnp.dot(u_ref[...], wproj[0:_INTD, :],
                          preferred_element_type=jnp.float32) + bproj[...])


def _proj_pair(s0, s1, u, v, wproj):
    n = u.shape[0]
    return pl.pallas_call(
        functools.partial(_proj_pair_body, s0, s1),
        out_shape=jax.ShapeDtypeStruct((n, _HD), jnp.float32),
    )(u, v, wproj)


_sc_mesh = plsc.VectorSubcoreMesh(core_axis_name="c", subcore_axis_name="s")


@functools.partial(
    pl.kernel,
    out_type=jax.ShapeDtypeStruct((_NT, _HD), jnp.float32),
    mesh=_sc_mesh,
    scratch_types=[
        pltpu.VMEM((_NC, 128), jnp.int32),
        pltpu.VMEM((_NC, 128), jnp.int32),
        pltpu.VMEM((_NC, 128), jnp.int32),
        pltpu.VMEM((_NC, 128), jnp.int32),
        pltpu.VMEM((_NC, 128), jnp.int32),
        pltpu.VMEM((_NC, 128), jnp.int32),
        pltpu.VMEM((_NC, 128), jnp.int32),
        pltpu.VMEM((_TB, _HD), jnp.float32),
        pltpu.VMEM((_TB, _HD), jnp.float32),
        pltpu.VMEM((_TB, _HD), jnp.float32),
        pltpu.SemaphoreType.DMA,
    ],
    compiler_params=pltpu.CompilerParams(use_tc_tiling_on_sc=False),
)
def _gather_sum(it_h, ii_h, ir_h, ig_h, iq_h, ta_h, tb_h, tc_h, x_h,
                it_v, ii_v, ir_v, ig_v, iq_v, cb_v, cc_v, a_v, b_v, c_v, sem):
    wid = lax.axis_index("s") * 2 + lax.axis_index("c")
    w_base = wid * _TPW

    def block_body(b, carry):
        base = w_base + b * _TB
        idx_copies = []
        for c in range(_NC):
            off = base + c * 128
            for (h, v) in ((it_h, it_v), (ii_h, ii_v), (ir_h, ir_v),
                           (ig_h, ig_v), (iq_h, iq_v)):
                idx_copies.append(
                    pltpu.async_copy(h.at[pl.ds(off, 128)], v.at[c], sem))
        for cp in idx_copies:
            cp.wait()

        for c in range(_NC):
            def comb_body(k, carry2, c=c):
                sl = (c, pl.ds(k * 16, 16))
                cb_v[sl] = ii_v[sl] * 3 + ir_v[sl]
                cc_v[sl] = ig_v[sl] * 14 + iq_v[sl]
                return carry2
            lax.fori_loop(0, 8, comb_body, 0, unroll=True)

        gathers = []
        for c in range(_NC):
            r = pl.ds(c * 128, 128)
            gathers.append(pltpu.async_copy(ta_h.at[it_v.at[c]], a_v.at[r], sem))
            gathers.append(pltpu.async_copy(tb_h.at[cb_v.at[c]], b_v.at[r], sem))
            gathers.append(pltpu.async_copy(tc_h.at[cc_v.at[c]], c_v.at[r], sem))
        for cp in gathers:
            cp.wait()

        def sum_body(t, carry2):
            for j in range(_HD // 16):
                sl = (t, pl.ds(j * 16, 16))
                plsc.addupdate(a_v.at[t, pl.ds(j * 16, 16)], b_v[sl] + c_v[sl])
            return carry2
        lax.fori_loop(0, _TB, sum_body, 0)

        pltpu.sync_copy(a_v, x_h.at[pl.ds(base, _TB), :])
        return carry

    lax.fori_loop(0, _NB, block_body, 0)


def _ln_body(x_ref, g_ref, b_ref, o_ref):
    x = x_ref[...]
    mu = jnp.mean(x, axis=1, keepdims=True)
    xc = x - mu
    var = jnp.mean(xc * xc, axis=1, keepdims=True)
    o_ref[...] = xc * lax.rsqrt(var + 1e-6) * g_ref[...] + b_ref[...]


def kernel(testId, assessmentItemID, KnowledgeTag, interaction, question_N,
           W_test, W_question, W_tag, W_inter, W_qN,
           W_proj, b_proj, ln_g, ln_b):
    f32 = jnp.float32
    i32 = jnp.int32
    ntest = W_test.shape[0]
    nq = W_question.shape[0]
    ntag = W_tag.shape[0]
    nqn = W_qN.shape[0]

    wq_rep = jnp.repeat(W_question, 3, axis=0)
    wi_flat = W_inter.reshape(3 * nq, _INTD)
    wtag_rep = jnp.repeat(W_tag, nqn, axis=0)
    wqn_tile = jnp.tile(W_qN, (ntag, 1))

    ta = pl.pallas_call(
        _proj_test_body,
        out_shape=jax.ShapeDtypeStruct((ntest, _HD), f32),
    )(W_test, W_proj, b_proj.reshape(1, _HD))
    tb = _proj_pair(_INTD, 3 * _INTD, wq_rep, wi_flat, W_proj)
    tc = _proj_pair(2 * _INTD, 4 * _INTD, wtag_rep, wqn_tile, W_proj)

    it = testId.reshape(_NT).astype(i32)
    ii = assessmentItemID.reshape(_NT).astype(i32)
    ir = interaction.reshape(_NT).astype(i32)
    ig = KnowledgeTag.reshape(_NT).astype(i32)
    iq = question_N.reshape(_NT).astype(i32)

    x = _gather_sum(it, ii, ir, ig, iq, ta, tb, tc)

    out = pl.pallas_call(
        _ln_body,
        grid=(_NT // 1024,),
        in_specs=[
            pl.BlockSpec((1024, _HD), lambda i: (i, 0)),
            pl.BlockSpec((1, _HD), lambda i: (0, 0)),
            pl.BlockSpec((1, _HD), lambda i: (0, 0)),
        ],
        out_specs=pl.BlockSpec((1024, _HD), lambda i: (i, 0)),
        out_shape=jax.ShapeDtypeStruct((_NT, _HD), f32),
    )(x, ln_g.reshape(1, _HD), ln_b.reshape(1, _HD))

    return out.reshape(_B, _L, _HD)


# speedup vs baseline: 16.5851x; 1.1968x over previous
import functools

import jax
import jax.numpy as jnp
from jax import lax
from jax.experimental import pallas as pl
from jax.experimental.pallas import tpu as pltpu
from jax.experimental.pallas import tpu_sc as plsc

_B, _L = 4096, 200
_HD = 64
_INTD = 32
_NT = _B * _L
_NW = 32
_TPW = _NT // _NW
_TB = 512
_NB = _TPW // _TB
_NC = _TB // 128


def _proj_pair_body(s0, s1, u_ref, v_ref, wproj, t_ref):
    f32 = jnp.float32
    p_u = wproj[s0:s0 + _INTD, :]
    p_v = wproj[s1:s1 + _INTD, :]
    t_ref[...] = (jnp.dot(u_ref[...], p_u, preferred_element_type=f32)
                  + jnp.dot(v_ref[...], p_v, preferred_element_type=f32))


def _proj_test_body(u_ref, wproj, bproj, t_ref):
    t_ref[...] = (jnp.dot(u_ref[...], wproj[0:_INTD, :],
                          preferred_element_type=jnp.float32) + bproj[...])


def _proj_pair(s0, s1, u, v, wproj):
    n = u.shape[0]
    return pl.pallas_call(
        functools.partial(_proj_pair_body, s0, s1),
        out_shape=jax.ShapeDtypeStruct((n, _HD), jnp.float32),
    )(u, v, wproj)


_sc_mesh = plsc.VectorSubcoreMesh(core_axis_name="c", subcore_axis_name="s")


@functools.partial(
    pl.kernel,
    out_type=jax.ShapeDtypeStruct((_NT, _HD), jnp.float32),
    mesh=_sc_mesh,
    scratch_types=[
        pltpu.VMEM((_NC, 128), jnp.int32),
        pltpu.VMEM((_NC, 128), jnp.int32),
        pltpu.VMEM((_NC, 128), jnp.int32),
        pltpu.VMEM((_NC, 128), jnp.int32),
        pltpu.VMEM((_NC, 128), jnp.int32),
        pltpu.VMEM((_NC, 128), jnp.int32),
        pltpu.VMEM((_NC, 128), jnp.int32),
        pltpu.VMEM((_TB, _HD), jnp.float32),
        pltpu.VMEM((_TB, _HD), jnp.float32),
        pltpu.VMEM((_TB, _HD), jnp.float32),
        pltpu.SemaphoreType.DMA,
    ],
    compiler_params=pltpu.CompilerParams(use_tc_tiling_on_sc=False),
)
def _gather_sum(it_h, ii_h, ir_h, ig_h, iq_h, ta_h, tb_h, tc_h, x_h,
                it_v, ii_v, ir_v, ig_v, iq_v, cb_v, cc_v, a_v, b_v, c_v, sem):
    wid = lax.axis_index("s") * 2 + lax.axis_index("c")
    w_base = wid * _TPW

    def block_body(b, carry):
        base = w_base + b * _TB
        idx_copies = []
        for c in range(_NC):
            off = base + c * 128
            for (h, v) in ((it_h, it_v), (ii_h, ii_v), (ir_h, ir_v),
                           (ig_h, ig_v), (iq_h, iq_v)):
                idx_copies.append(
                    pltpu.async_copy(h.at[pl.ds(off, 128)], v.at[c], sem))
        for cp in idx_copies:
            cp.wait()

        for c in range(_NC):
            def comb_body(k, carry2, c=c):
                sl = (c, pl.ds(k * 16, 16))
                cb_v[sl] = ii_v[sl] * 3 + ir_v[sl]
                cc_v[sl] = ig_v[sl] * 14 + iq_v[sl]
                return carry2
            lax.fori_loop(0, 8, comb_body, 0, unroll=True)

        gathers = []
        for c in range(_NC):
            r = pl.ds(c * 128, 128)
            gathers.append(pltpu.async_copy(ta_h.at[it_v.at[c]], a_v.at[r], sem))
            gathers.append(pltpu.async_copy(tb_h.at[cb_v.at[c]], b_v.at[r], sem))
            gathers.append(pltpu.async_copy(tc_h.at[cc_v.at[c]], c_v.at[r], sem))
        for cp in gathers:
            cp.wait()

        def sum_body(t, carry2):
            for j in range(_HD // 16):
                sl = (t, pl.ds(j * 16, 16))
                plsc.addupdate(a_v.at[t, pl.ds(j * 16, 16)], b_v[sl] + c_v[sl])
            return carry2
        lax.fori_loop(0, _TB, sum_body, 0)

        pltpu.sync_copy(a_v, x_h.at[pl.ds(base, _TB), :])
        return carry

    lax.fori_loop(0, _NB, block_body, 0)


def _ln_body(x_ref, g_ref, b_ref, o_ref):
    x = x_ref[...]
    g = g_ref[...]
    bb = b_ref[...]
    for h in range(2):
        sl = (slice(None), slice(h * _HD, (h + 1) * _HD))
        xh = x[sl]
        mu = jnp.mean(xh, axis=1, keepdims=True)
        xc = xh - mu
        var = jnp.mean(xc * xc, axis=1, keepdims=True)
        o_ref[sl] = xc * lax.rsqrt(var + 1e-6) * g[sl] + bb[sl]


def kernel(testId, assessmentItemID, KnowledgeTag, interaction, question_N,
           W_test, W_question, W_tag, W_inter, W_qN,
           W_proj, b_proj, ln_g, ln_b):
    f32 = jnp.float32
    i32 = jnp.int32
    ntest = W_test.shape[0]
    nq = W_question.shape[0]
    ntag = W_tag.shape[0]
    nqn = W_qN.shape[0]

    wq_rep = jnp.broadcast_to(W_question[:, None, :],
                              (nq, 3, _INTD)).reshape(3 * nq, _INTD)
    wi_flat = W_inter.reshape(3 * nq, _INTD)
    wtag_rep = jnp.broadcast_to(W_tag[:, None, :],
                                (ntag, nqn, _INTD)).reshape(ntag * nqn, _INTD)
    wqn_tile = jnp.broadcast_to(W_qN[None, :, :],
                                (ntag, nqn, _INTD)).reshape(ntag * nqn, _INTD)

    ta = pl.pallas_call(
        _proj_test_body,
        out_shape=jax.ShapeDtypeStruct((ntest, _HD), f32),
    )(W_test, W_proj, b_proj.reshape(1, _HD))
    tb = _proj_pair(_INTD, 3 * _INTD, wq_rep, wi_flat, W_proj)
    tc = _proj_pair(2 * _INTD, 4 * _INTD, wtag_rep, wqn_tile, W_proj)

    it = testId.reshape(_NT).astype(i32)
    ii = assessmentItemID.reshape(_NT).astype(i32)
    ir = interaction.reshape(_NT).astype(i32)
    ig = KnowledgeTag.reshape(_NT).astype(i32)
    iq = question_N.reshape(_NT).astype(i32)

    x = _gather_sum(it, ii, ir, ig, iq, ta, tb, tc)

    x2 = x.reshape(_NT // 2, 2 * _HD)
    g2 = jnp.broadcast_to(ln_g[None, :], (2, _HD)).reshape(1, 2 * _HD)
    b2 = jnp.broadcast_to(ln_b[None, :], (2, _HD)).reshape(1, 2 * _HD)
    out = pl.pallas_call(
        _ln_body,
        grid=(_NT // 2 // 2048,),
        in_specs=[
            pl.BlockSpec((2048, 2 * _HD), lambda i: (i, 0)),
            pl.BlockSpec((1, 2 * _HD), lambda i: (0, 0)),
            pl.BlockSpec((1, 2 * _HD), lambda i: (0, 0)),
        ],
        out_specs=pl.BlockSpec((2048, 2 * _HD), lambda i: (i, 0)),
        out_shape=jax.ShapeDtypeStruct((_NT // 2, 2 * _HD), f32),
    )(x2, g2, b2)

    return out.reshape(_B, _L, _HD)


# speedup vs baseline: 18.4169x; 1.1104x over previous
import functools

import jax
import jax.numpy as jnp
from jax import lax
from jax.experimental import pallas as pl
from jax.experimental.pallas import tpu as pltpu
from jax.experimental.pallas import tpu_sc as plsc

_B, _L = 4096, 200
_HD = 64
_INTD = 32
_NT = _B * _L
_NW = 32
_TPW = _NT // _NW
_TB = 512
_NB = _TPW // _TB
_NC = _TB // 128


def _proj_pair_body(s0, s1, u_ref, v_ref, wproj, t_ref):
    f32 = jnp.float32
    p_u = wproj[s0:s0 + _INTD, :]
    p_v = wproj[s1:s1 + _INTD, :]
    t_ref[...] = (jnp.dot(u_ref[...], p_u, preferred_element_type=f32)
                  + jnp.dot(v_ref[...], p_v, preferred_element_type=f32))


def _proj_test_body(u_ref, wproj, bproj, t_ref):
    t_ref[...] = (jnp.dot(u_ref[...], wproj[0:_INTD, :],
                          preferred_element_type=jnp.float32) + bproj[...])


def _proj_pair(s0, s1, u, v, wproj):
    n = u.shape[0]
    return pl.pallas_call(
        functools.partial(_proj_pair_body, s0, s1),
        out_shape=jax.ShapeDtypeStruct((n, _HD), jnp.float32),
    )(u, v, wproj)


_sc_mesh = plsc.VectorSubcoreMesh(core_axis_name="c", subcore_axis_name="s")


@functools.partial(
    pl.kernel,
    out_type=jax.ShapeDtypeStruct((_NT, 2 * _HD), jnp.float32),
    mesh=_sc_mesh,
    scratch_types=[
        pltpu.VMEM((_NC, 128), jnp.int32),
        pltpu.VMEM((_NC, 128), jnp.int32),
        pltpu.VMEM((_NC, 128), jnp.int32),
        pltpu.VMEM((_NC, 128), jnp.int32),
        pltpu.VMEM((_NC, 128), jnp.int32),
        pltpu.VMEM((_NC, 128), jnp.int32),
        pltpu.VMEM((_NC, 128), jnp.int32),
        pltpu.VMEM((_TB, _HD), jnp.float32),
        pltpu.VMEM((_TB, _HD), jnp.float32),
        pltpu.VMEM((_TB, _HD), jnp.float32),
        pltpu.SemaphoreType.DMA,
    ],
    compiler_params=pltpu.CompilerParams(use_tc_tiling_on_sc=False),
)
def _gather_sum(it_h, ii_h, ir_h, ig_h, iq_h, ta_h, tb_h, tc_h, x_h,
                it_v, ii_v, ir_v, ig_v, iq_v, cb_v, cc_v, a_v, b_v, c_v, sem):
    wid = lax.axis_index("s") * 2 + lax.axis_index("c")
    w_base = wid * _TPW

    def block_body(b, carry):
        base = w_base + b * _TB
        idx_copies = []
        for c in range(_NC):
            off = base + c * 128
            for (h, v) in ((it_h, it_v), (ii_h, ii_v), (ir_h, ir_v),
                           (ig_h, ig_v), (iq_h, iq_v)):
                idx_copies.append(
                    pltpu.async_copy(h.at[pl.ds(off, 128)], v.at[c], sem))
        for cp in idx_copies:
            cp.wait()

        for c in range(_NC):
            def comb_body(k, carry2, c=c):
                sl = (c, pl.ds(k * 16, 16))
                cb_v[sl] = ii_v[sl] * 3 + ir_v[sl]
                cc_v[sl] = ig_v[sl] * 14 + iq_v[sl]
                return carry2
            lax.fori_loop(0, 8, comb_body, 0, unroll=True)

        gathers = []
        for c in range(_NC):
            r = pl.ds(c * 128, 128)
            gathers.append(pltpu.async_copy(ta_h.at[it_v.at[c]], a_v.at[r], sem))
            gathers.append(pltpu.async_copy(tb_h.at[cb_v.at[c]], b_v.at[r], sem))
            gathers.append(pltpu.async_copy(tc_h.at[cc_v.at[c]], c_v.at[r], sem))
        for cp in gathers:
            cp.wait()

        def sum_body(t, carry2):
            for j in range(_HD // 16):
                sl = (t, pl.ds(j * 16, 16))
                plsc.addupdate(a_v.at[t, pl.ds(j * 16, 16)], b_v[sl] + c_v[sl])
            return carry2
        lax.fori_loop(0, _TB, sum_body, 0)

        pltpu.sync_copy(a_v, x_h.at[pl.ds(base, _TB), pl.ds(0, _HD)])
        return carry

    lax.fori_loop(0, _NB, block_body, 0)


_LNB = 16


def _ln_body(x_ref, g_ref, b_ref, o_ref):
    x = x_ref[:, 0:_HD]
    mu = jnp.mean(x, axis=1, keepdims=True)
    xc = x - mu
    var = jnp.mean(xc * xc, axis=1, keepdims=True)
    y = xc * lax.rsqrt(var + 1e-6) * g_ref[...] + b_ref[...]
    for h in range(_LNB):
        o_ref[h] = y[h * _L:(h + 1) * _L, :]


def kernel(testId, assessmentItemID, KnowledgeTag, interaction, question_N,
           W_test, W_question, W_tag, W_inter, W_qN,
           W_proj, b_proj, ln_g, ln_b):
    f32 = jnp.float32
    i32 = jnp.int32
    ntest = W_test.shape[0]
    nq = W_question.shape[0]
    ntag = W_tag.shape[0]
    nqn = W_qN.shape[0]

    wq_rep = jnp.broadcast_to(W_question[:, None, :],
                              (nq, 3, _INTD)).reshape(3 * nq, _INTD)
    wi_flat = W_inter.reshape(3 * nq, _INTD)
    wtag_rep = jnp.broadcast_to(W_tag[:, None, :],
                                (ntag, nqn, _INTD)).reshape(ntag * nqn, _INTD)
    wqn_tile = jnp.broadcast_to(W_qN[None, :, :],
                                (ntag, nqn, _INTD)).reshape(ntag * nqn, _INTD)

    ta = pl.pallas_call(
        _proj_test_body,
        out_shape=jax.ShapeDtypeStruct((ntest, _HD), f32),
    )(W_test, W_proj, b_proj.reshape(1, _HD))
    tb = _proj_pair(_INTD, 3 * _INTD, wq_rep, wi_flat, W_proj)
    tc = _proj_pair(2 * _INTD, 4 * _INTD, wtag_rep, wqn_tile, W_proj)

    it = testId.reshape(_NT).astype(i32)
    ii = assessmentItemID.reshape(_NT).astype(i32)
    ir = interaction.reshape(_NT).astype(i32)
    ig = KnowledgeTag.reshape(_NT).astype(i32)
    iq = question_N.reshape(_NT).astype(i32)

    x = _gather_sum(it, ii, ir, ig, iq, ta, tb, tc)

    out = pl.pallas_call(
        _ln_body,
        grid=(_B // _LNB,),
        in_specs=[
            pl.BlockSpec((_LNB * _L, 2 * _HD), lambda i: (i, 0)),
            pl.BlockSpec((1, _HD), lambda i: (0, 0)),
            pl.BlockSpec((1, _HD), lambda i: (0, 0)),
        ],
        out_specs=pl.BlockSpec((_LNB, _L, _HD), lambda i: (i, 0, 0)),
        out_shape=jax.ShapeDtypeStruct((_B, _L, _HD), f32),
    )(x, ln_g.reshape(1, _HD), ln_b.reshape(1, _HD))

    return out


# speedup vs baseline: 20.4356x; 1.1096x over previous
import functools

import jax
import jax.numpy as jnp
from jax import lax
from jax.experimental import pallas as pl
from jax.experimental.pallas import tpu as pltpu
from jax.experimental.pallas import tpu_sc as plsc

_B, _L = 4096, 200
_HD = 64
_INTD = 32
_NT = _B * _L
_NW = 32
_TPW = _NT // _NW
_TB = 256
_NB = _TPW // _TB
_NC = _TB // 128
_NCH = _NT // 128


def _proj_pair_body(s0, s1, u_ref, v_ref, wproj, t_ref):
    f32 = jnp.float32
    p_u = wproj[s0:s0 + _INTD, :]
    p_v = wproj[s1:s1 + _INTD, :]
    t_ref[...] = (jnp.dot(u_ref[...], p_u, preferred_element_type=f32)
                  + jnp.dot(v_ref[...], p_v, preferred_element_type=f32))


def _proj_test_body(u_ref, wproj, bproj, t_ref):
    t_ref[...] = (jnp.dot(u_ref[...], wproj[0:_INTD, :],
                          preferred_element_type=jnp.float32) + bproj[...])


def _proj_pair(s0, s1, u, v, wproj):
    n = u.shape[0]
    return pl.pallas_call(
        functools.partial(_proj_pair_body, s0, s1),
        out_shape=jax.ShapeDtypeStruct((n, _HD), jnp.float32),
    )(u, v, wproj)


_sc_mesh = plsc.VectorSubcoreMesh(core_axis_name="c", subcore_axis_name="s")


@functools.partial(
    pl.kernel,
    out_type=jax.ShapeDtypeStruct((_NT, 2 * _HD), jnp.float32),
    mesh=_sc_mesh,
    scratch_types=[
        pltpu.VMEM((2, _NC, 5, 128), jnp.int32),
        pltpu.VMEM((2, _NC, 128), jnp.int32),
        pltpu.VMEM((2, _NC, 128), jnp.int32),
        pltpu.VMEM((2, _TB, _HD), jnp.float32),
        pltpu.VMEM((2, _TB, _HD), jnp.float32),
        pltpu.VMEM((2, _TB, _HD), jnp.float32),
        pltpu.SemaphoreType.DMA,
        pltpu.SemaphoreType.DMA,
        pltpu.SemaphoreType.DMA,
        pltpu.SemaphoreType.DMA,
        pltpu.SemaphoreType.DMA,
        pltpu.SemaphoreType.DMA,
    ],
    compiler_params=pltpu.CompilerParams(use_tc_tiling_on_sc=False),
)
def _gather_sum(idx5_h, ta_h, tb_h, tc_h, x_h,
                i_v, cb_v, cc_v, a_v, b_v, c_v,
                si0, si1, sg0, sg1, so0, so1):
    wid = lax.axis_index("s") * 2 + lax.axis_index("c")
    w_chunk = wid * (_TPW // 128)
    w_base = wid * _TPW
    sem_i = (si0, si1)
    sem_g = (sg0, sg1)
    sem_o = (so0, so1)

    def fire_idx(g, s):
        for c in range(_NC):
            pltpu.async_copy(idx5_h.at[w_chunk + g * _NC + c],
                             i_v.at[s, c], sem_i[s])

    def wait_idx(s):
        for c in range(_NC):
            pltpu.make_async_copy(idx5_h.at[w_chunk],
                                  i_v.at[s, c], sem_i[s]).wait()

    def comb(s):
        for c in range(_NC):
            def body(k, carry, c=c):
                sl = pl.ds(k * 16, 16)
                cb_v[s, c, sl] = i_v[s, c, 1, sl] * 3 + i_v[s, c, 2, sl]
                cc_v[s, c, sl] = i_v[s, c, 3, sl] * 14 + i_v[s, c, 4, sl]
                return carry
            lax.fori_loop(0, 128 // 16, body, 0, unroll=True)

    def fire_gathers(s):
        for c in range(_NC):
            r = pl.ds(c * 128, 128)
            pltpu.async_copy(ta_h.at[i_v.at[s, c, 0]], a_v.at[s, r, :], sem_g[s])
            pltpu.async_copy(tb_h.at[cb_v.at[s, c]], b_v.at[s, r, :], sem_g[s])
            pltpu.async_copy(tc_h.at[cc_v.at[s, c]], c_v.at[s, r, :], sem_g[s])

    def wait_gathers(s):
        for c in range(_NC):
            r = pl.ds(c * 128, 128)
            pltpu.make_async_copy(ta_h.at[i_v.at[s, c, 0]],
                                  a_v.at[s, r, :], sem_g[s]).wait()
            pltpu.make_async_copy(tb_h.at[cb_v.at[s, c]],
                                  b_v.at[s, r, :], sem_g[s]).wait()
            pltpu.make_async_copy(tc_h.at[cc_v.at[s, c]],
                                  c_v.at[s, r, :], sem_g[s]).wait()

    def sum_rows(s):
        def body(t, carry):
            for j in range(_HD // 16):
                sl = pl.ds(j * 16, 16)
                plsc.addupdate(a_v.at[s, t, sl], b_v[s, t, sl] + c_v[s, t, sl])
            return carry
        lax.fori_loop(0, _TB, body, 0, unroll=4)

    def fire_out(g, s):
        pltpu.async_copy(a_v.at[s],
                         x_h.at[pl.ds(w_base + g * _TB, _TB), pl.ds(0, _HD)],
                         sem_o[s])

    def wait_out(s):
        pltpu.make_async_copy(a_v.at[s],
                              x_h.at[pl.ds(w_base, _TB), pl.ds(0, _HD)],
                              sem_o[s]).wait()

    def phase(g, s, out_wait, fire_next=True):
        wait_idx(s)
        comb(s)
        if out_wait:
            wait_out(s)
        fire_gathers(s)
        if fire_next:
            fire_idx(g + 1, 1 - s)
        wait_gathers(1 - s)
        sum_rows(1 - s)
        fire_out(g - 1, 1 - s)

    fire_idx(0, 0)
    wait_idx(0)
    comb(0)
    fire_gathers(0)
    fire_idx(1, 1)
    phase(1, 1, out_wait=False)
    phase(2, 0, out_wait=True)

    def pair(k, carry):
        g = 3 + 2 * k
        phase(g, 1, True)
        phase(g + 1, 0, True)
        return carry
    lax.fori_loop(0, (_NB - 4) // 2, pair, 0)

    phase(_NB - 1, 1, out_wait=True, fire_next=False)
    wait_gathers(1)
    sum_rows(1)
    fire_out(_NB - 1, 1)
    wait_out(0)
    wait_out(1)


_LNB = 16


def _ln_body(x_ref, g_ref, b_ref, o_ref):
    x = x_ref[:, 0:_HD]
    mu = jnp.mean(x, axis=1, keepdims=True)
    xc = x - mu
    var = jnp.mean(xc * xc, axis=1, keepdims=True)
    y = xc * lax.rsqrt(var + 1e-6) * g_ref[...] + b_ref[...]
    for h in range(_LNB):
        o_ref[h] = y[h * _L:(h + 1) * _L, :]


def kernel(testId, assessmentItemID, KnowledgeTag, interaction, question_N,
           W_test, W_question, W_tag, W_inter, W_qN,
           W_proj, b_proj, ln_g, ln_b):
    f32 = jnp.float32
    i32 = jnp.int32
    ntest = W_test.shape[0]
    nq = W_question.shape[0]
    ntag = W_tag.shape[0]
    nqn = W_qN.shape[0]

    wq_rep = jnp.broadcast_to(W_question[:, None, :],
                              (nq, 3, _INTD)).reshape(3 * nq, _INTD)
    wi_flat = W_inter.reshape(3 * nq, _INTD)
    wtag_rep = jnp.broadcast_to(W_tag[:, None, :],
                                (ntag, nqn, _INTD)).reshape(ntag * nqn, _INTD)
    wqn_tile = jnp.broadcast_to(W_qN[None, :, :],
                                (ntag, nqn, _INTD)).reshape(ntag * nqn, _INTD)

    ta = pl.pallas_call(
        _proj_test_body,
        out_shape=jax.ShapeDtypeStruct((ntest, _HD), f32),
    )(W_test, W_proj, b_proj.reshape(1, _HD))
    tb = _proj_pair(_INTD, 3 * _INTD, wq_rep, wi_flat, W_proj)
    tc = _proj_pair(2 * _INTD, 4 * _INTD, wtag_rep, wqn_tile, W_proj)

    idx5 = jnp.stack(
        [a.reshape(_NCH, 128).astype(i32)
         for a in (testId, assessmentItemID, interaction,
                   KnowledgeTag, question_N)],
        axis=1)

    x = _gather_sum(idx5, ta, tb, tc)

    out = pl.pallas_call(
        _ln_body,
        grid=(_B // _LNB,),
        in_specs=[
            pl.BlockSpec((_LNB * _L, 2 * _HD), lambda i: (i, 0)),
            pl.BlockSpec((1, _HD), lambda i: (0, 0)),
            pl.BlockSpec((1, _HD), lambda i: (0, 0)),
        ],
        out_specs=pl.BlockSpec((_LNB, _L, _HD), lambda i: (i, 0, 0)),
        out_shape=jax.ShapeDtypeStruct((_B, _L, _HD), f32),
    )(x, ln_g.reshape(1, _HD), ln_b.reshape(1, _HD))

    return out


# speedup vs baseline: 21.4393x; 1.0491x over previous
import functools

import jax
import jax.numpy as jnp
from jax import lax
from jax.experimental import pallas as pl
from jax.experimental.pallas import tpu as pltpu
from jax.experimental.pallas import tpu_sc as plsc

_B, _L = 4096, 200
_HD = 64
_INTD = 32
_NT = _B * _L
_NW = 32
_TPW = _NT // _NW
_TB = 256
_NB = _TPW // _TB
_NC = _TB // 128
_NCH = _NT // 128


def _prep_body(u_ref, v_ref, wtest_ref, wproj, bproj, tb, tc, ta):
    f32 = jnp.float32
    p_b = jnp.concatenate([wproj[_INTD:2 * _INTD, :],
                           wproj[3 * _INTD:4 * _INTD, :]], axis=0)
    p_c = jnp.concatenate([wproj[2 * _INTD:3 * _INTD, :],
                           wproj[4 * _INTD:5 * _INTD, :]], axis=0)
    tb[...] = jnp.dot(u_ref[...], p_b, preferred_element_type=f32)
    tc[...] = jnp.dot(v_ref[...], p_c, preferred_element_type=f32)
    ta[...] = (jnp.dot(wtest_ref[...], wproj[0:_INTD, :],
                       preferred_element_type=f32) + bproj[...])


_sc_mesh = plsc.VectorSubcoreMesh(core_axis_name="c", subcore_axis_name="s")


@functools.partial(
    pl.kernel,
    out_type=jax.ShapeDtypeStruct((_NT, 2 * _HD), jnp.float32),
    mesh=_sc_mesh,
    scratch_types=[
        pltpu.VMEM((2, _NC, 5, 128), jnp.int32),
        pltpu.VMEM((2, _NC, 128), jnp.int32),
        pltpu.VMEM((2, _NC, 128), jnp.int32),
        pltpu.VMEM((2, _TB, _HD), jnp.float32),
        pltpu.VMEM((2, _TB, _HD), jnp.float32),
        pltpu.VMEM((2, _TB, _HD), jnp.float32),
        pltpu.SemaphoreType.DMA,
        pltpu.SemaphoreType.DMA,
        pltpu.SemaphoreType.DMA,
        pltpu.SemaphoreType.DMA,
        pltpu.SemaphoreType.DMA,
        pltpu.SemaphoreType.DMA,
    ],
    compiler_params=pltpu.CompilerParams(use_tc_tiling_on_sc=False),
)
def _gather_sum(idx5_h, ta_h, tb_h, tc_h, x_h,
                i_v, cb_v, cc_v, a_v, b_v, c_v,
                si0, si1, sg0, sg1, so0, so1):
    wid = lax.axis_index("s") * 2 + lax.axis_index("c")
    w_chunk = wid * (_TPW // 128)
    w_base = wid * _TPW
    sem_i = (si0, si1)
    sem_g = (sg0, sg1)
    sem_o = (so0, so1)

    def fire_idx(g, s):
        for c in range(_NC):
            pltpu.async_copy(idx5_h.at[w_chunk + g * _NC + c],
                             i_v.at[s, c], sem_i[s])

    def wait_idx(s):
        for c in range(_NC):
            pltpu.make_async_copy(idx5_h.at[w_chunk],
                                  i_v.at[s, c], sem_i[s]).wait()

    def comb(s):
        for c in range(_NC):
            def body(k, carry, c=c):
                sl = pl.ds(k * 16, 16)
                cb_v[s, c, sl] = i_v[s, c, 1, sl] * 3 + i_v[s, c, 2, sl]
                cc_v[s, c, sl] = i_v[s, c, 3, sl] * 14 + i_v[s, c, 4, sl]
                return carry
            lax.fori_loop(0, 128 // 16, body, 0, unroll=True)

    def fire_gathers(s):
        for c in range(_NC):
            r = pl.ds(c * 128, 128)
            pltpu.async_copy(ta_h.at[i_v.at[s, c, 0]], a_v.at[s, r, :], sem_g[s])
            pltpu.async_copy(tb_h.at[cb_v.at[s, c]], b_v.at[s, r, :], sem_g[s])
            pltpu.async_copy(tc_h.at[cc_v.at[s, c]], c_v.at[s, r, :], sem_g[s])

    def wait_gathers(s):
        for c in range(_NC):
            r = pl.ds(c * 128, 128)
            pltpu.make_async_copy(ta_h.at[i_v.at[s, c, 0]],
                                  a_v.at[s, r, :], sem_g[s]).wait()
            pltpu.make_async_copy(tb_h.at[cb_v.at[s, c]],
                                  b_v.at[s, r, :], sem_g[s]).wait()
            pltpu.make_async_copy(tc_h.at[cc_v.at[s, c]],
                                  c_v.at[s, r, :], sem_g[s]).wait()

    def sum_rows(s):
        def body(t, carry):
            for j in range(_HD // 16):
                sl = pl.ds(j * 16, 16)
                plsc.addupdate(a_v.at[s, t, sl], b_v[s, t, sl] + c_v[s, t, sl])
            return carry
        lax.fori_loop(0, _TB, body, 0, unroll=4)

    def fire_out(g, s):
        pltpu.async_copy(a_v.at[s],
                         x_h.at[pl.ds(w_base + g * _TB, _TB), pl.ds(0, _HD)],
                         sem_o[s])

    def wait_out(s):
        pltpu.make_async_copy(a_v.at[s],
                              x_h.at[pl.ds(w_base, _TB), pl.ds(0, _HD)],
                              sem_o[s]).wait()

    def phase(g, s, out_wait, fire_next=True):
        wait_idx(s)
        comb(s)
        if out_wait:
            wait_out(s)
        fire_gathers(s)
        if fire_next:
            fire_idx(g + 1, 1 - s)
        wait_gathers(1 - s)
        sum_rows(1 - s)
        fire_out(g - 1, 1 - s)

    fire_idx(0, 0)
    wait_idx(0)
    comb(0)
    fire_gathers(0)
    fire_idx(1, 1)
    phase(1, 1, out_wait=False)
    phase(2, 0, out_wait=True)

    def pair(k, carry):
        g = 3 + 2 * k
        phase(g, 1, True)
        phase(g + 1, 0, True)
        return carry
    lax.fori_loop(0, (_NB - 4) // 2, pair, 0)

    phase(_NB - 1, 1, out_wait=True, fire_next=False)
    wait_gathers(1)
    sum_rows(1)
    fire_out(_NB - 1, 1)
    wait_out(0)
    wait_out(1)


_LNB = 32


def _ln_body(x_ref, g_ref, b_ref, o_ref):
    x = x_ref[:, 0:_HD]
    mu = jnp.mean(x, axis=1, keepdims=True)
    xc = x - mu
    var = jnp.mean(xc * xc, axis=1, keepdims=True)
    y = xc * lax.rsqrt(var + 1e-6) * g_ref[...] + b_ref[...]
    for h in range(_LNB):
        o_ref[h] = y[h * _L:(h + 1) * _L, :]


def kernel(testId, assessmentItemID, KnowledgeTag, interaction, question_N,
           W_test, W_question, W_tag, W_inter, W_qN,
           W_proj, b_proj, ln_g, ln_b):
    f32 = jnp.float32
    i32 = jnp.int32
    ntest = W_test.shape[0]
    nq = W_question.shape[0]
    ntag = W_tag.shape[0]
    nqn = W_qN.shape[0]

    wq_rep = jnp.broadcast_to(W_question[:, None, :],
                              (nq, 3, _INTD)).reshape(3 * nq, _INTD)
    wi_flat = W_inter.reshape(3 * nq, _INTD)
    wtag_rep = jnp.broadcast_to(W_tag[:, None, :],
                                (ntag, nqn, _INTD)).reshape(ntag * nqn, _INTD)
    wqn_tile = jnp.broadcast_to(W_qN[None, :, :],
                                (ntag, nqn, _INTD)).reshape(ntag * nqn, _INTD)
    u = jnp.concatenate([wq_rep, wi_flat], axis=1)
    v = jnp.concatenate([wtag_rep, wqn_tile], axis=1)

    tb, tc, ta = pl.pallas_call(
        _prep_body,
        out_shape=(
            jax.ShapeDtypeStruct((3 * nq, _HD), f32),
            jax.ShapeDtypeStruct((ntag * nqn, _HD), f32),
            jax.ShapeDtypeStruct((ntest, _HD), f32),
        ),
    )(u, v, W_test, W_proj, b_proj.reshape(1, _HD))

    idx5 = jnp.stack(
        [a.reshape(_NCH, 128).astype(i32)
         for a in (testId, assessmentItemID, interaction,
                   KnowledgeTag, question_N)],
        axis=1)

    x = _gather_sum(idx5, ta, tb, tc)

    out = pl.pallas_call(
        _ln_body,
        grid=(_B // _LNB,),
        in_specs=[
            pl.BlockSpec((_LNB * _L, 2 * _HD), lambda i: (i, 0)),
            pl.BlockSpec((1, _HD), lambda i: (0, 0)),
            pl.BlockSpec((1, _HD), lambda i: (0, 0)),
        ],
        out_specs=pl.BlockSpec((_LNB, _L, _HD), lambda i: (i, 0, 0)),
        out_shape=jax.ShapeDtypeStruct((_B, _L, _HD), f32),
    )(x, ln_g.reshape(1, _HD), ln_b.reshape(1, _HD))

    return out


# speedup vs baseline: 22.9347x; 1.0697x over previous
import functools

import jax
import jax.numpy as jnp
from jax import lax
from jax.experimental import pallas as pl
from jax.experimental.pallas import tpu as pltpu
from jax.experimental.pallas import tpu_sc as plsc

_B, _L = 4096, 200
_HD = 64
_INTD = 32
_NT = _B * _L
_NW = 32
_TPW = _NT // _NW
_TB = 256
_NB = _TPW // _TB
_NC = _TB // 128
_NCH = _NT // 128


def _prep_body(u_ref, v_ref, wtest_ref, wproj, bproj, tb, tc, ta):
    f32 = jnp.float32
    p_b = jnp.concatenate([wproj[_INTD:2 * _INTD, :],
                           wproj[3 * _INTD:4 * _INTD, :]], axis=0)
    p_c = jnp.concatenate([wproj[2 * _INTD:3 * _INTD, :],
                           wproj[4 * _INTD:5 * _INTD, :]], axis=0)
    tb[...] = jnp.dot(u_ref[...], p_b, preferred_element_type=f32)
    tc[...] = jnp.dot(v_ref[...], p_c, preferred_element_type=f32)
    ta[...] = (jnp.dot(wtest_ref[...], wproj[0:_INTD, :],
                       preferred_element_type=f32) + bproj[...])


_sc_mesh = plsc.VectorSubcoreMesh(core_axis_name="c", subcore_axis_name="s")


@functools.partial(
    pl.kernel,
    out_type=jax.ShapeDtypeStruct((_NT, 2 * _HD), jnp.float32),
    mesh=_sc_mesh,
    scratch_types=[
        pltpu.VMEM((2, 5, _NC, 128), jnp.int32),
        pltpu.VMEM((2, _NC, 128), jnp.int32),
        pltpu.VMEM((2, _NC, 128), jnp.int32),
        pltpu.VMEM((2, _TB, _HD), jnp.float32),
        pltpu.VMEM((2, _TB, _HD), jnp.float32),
        pltpu.VMEM((2, _TB, _HD), jnp.float32),
        pltpu.SemaphoreType.DMA,
        pltpu.SemaphoreType.DMA,
        pltpu.SemaphoreType.DMA,
        pltpu.SemaphoreType.DMA,
        pltpu.SemaphoreType.DMA,
        pltpu.SemaphoreType.DMA,
    ],
    compiler_params=pltpu.CompilerParams(use_tc_tiling_on_sc=False),
)
def _gather_sum(it_h, ii_h, ir_h, ig_h, iq_h, ta_h, tb_h, tc_h, x_h,
                i_v, cb_v, cc_v, a_v, b_v, c_v,
                si0, si1, sg0, sg1, so0, so1):
    wid = lax.axis_index("s") * 2 + lax.axis_index("c")
    w_base = wid * _TPW
    sem_i = (si0, si1)
    sem_g = (sg0, sg1)
    sem_o = (so0, so1)
    idx_hs = (it_h, ii_h, ir_h, ig_h, iq_h)

    def fire_idx(g, s):
        base = w_base + g * _TB
        for a, h in enumerate(idx_hs):
            for c in range(_NC):
                pltpu.async_copy(h.at[pl.ds(base + c * 128, 128)],
                                 i_v.at[s, a, c], sem_i[s])

    def wait_idx(s):
        for a, h in enumerate(idx_hs):
            for c in range(_NC):
                pltpu.make_async_copy(h.at[pl.ds(w_base, 128)],
                                      i_v.at[s, a, c], sem_i[s]).wait()

    def comb(s):
        for c in range(_NC):
            def body(k, carry, c=c):
                sl = pl.ds(k * 16, 16)
                cb_v[s, c, sl] = i_v[s, 1, c, sl] * 3 + i_v[s, 2, c, sl]
                cc_v[s, c, sl] = i_v[s, 3, c, sl] * 14 + i_v[s, 4, c, sl]
                return carry
            lax.fori_loop(0, 128 // 16, body, 0, unroll=True)

    def fire_gathers(s):
        for c in range(_NC):
            r = pl.ds(c * 128, 128)
            pltpu.async_copy(ta_h.at[i_v.at[s, 0, c]], a_v.at[s, r, :], sem_g[s])
            pltpu.async_copy(tb_h.at[cb_v.at[s, c]], b_v.at[s, r, :], sem_g[s])
            pltpu.async_copy(tc_h.at[cc_v.at[s, c]], c_v.at[s, r, :], sem_g[s])

    def wait_gathers(s):
        for c in range(_NC):
            r = pl.ds(c * 128, 128)
            pltpu.make_async_copy(ta_h.at[i_v.at[s, 0, c]],
                                  a_v.at[s, r, :], sem_g[s]).wait()
            pltpu.make_async_copy(tb_h.at[cb_v.at[s, c]],
                                  b_v.at[s, r, :], sem_g[s]).wait()
            pltpu.make_async_copy(tc_h.at[cc_v.at[s, c]],
                                  c_v.at[s, r, :], sem_g[s]).wait()

    def sum_rows(s):
        def body(t, carry):
            for j in range(_HD // 16):
                sl = pl.ds(j * 16, 16)
                plsc.addupdate(a_v.at[s, t, sl], b_v[s, t, sl] + c_v[s, t, sl])
            return carry
        lax.fori_loop(0, _TB, body, 0, unroll=4)

    def fire_out(g, s):
        pltpu.async_copy(a_v.at[s],
                         x_h.at[pl.ds(w_base + g * _TB, _TB), pl.ds(0, _HD)],
                         sem_o[s])

    def wait_out(s):
        pltpu.make_async_copy(a_v.at[s],
                              x_h.at[pl.ds(w_base, _TB), pl.ds(0, _HD)],
                              sem_o[s]).wait()

    def phase(g, s, out_wait, fire_next=True):
        wait_idx(s)
        comb(s)
        if out_wait:
            wait_out(s)
        fire_gathers(s)
        if fire_next:
            fire_idx(g + 1, 1 - s)
        wait_gathers(1 - s)
        sum_rows(1 - s)
        fire_out(g - 1, 1 - s)

    fire_idx(0, 0)
    wait_idx(0)
    comb(0)
    fire_gathers(0)
    fire_idx(1, 1)
    phase(1, 1, out_wait=False)
    phase(2, 0, out_wait=True)

    def pair(k, carry):
        g = 3 + 2 * k
        phase(g, 1, True)
        phase(g + 1, 0, True)
        return carry
    lax.fori_loop(0, (_NB - 4) // 2, pair, 0)

    phase(_NB - 1, 1, out_wait=True, fire_next=False)
    wait_gathers(1)
    sum_rows(1)
    fire_out(_NB - 1, 1)
    wait_out(0)
    wait_out(1)


_LNB = 32


def _ln_body(x_ref, g_ref, b_ref, o_ref):
    x = x_ref[:, 0:_HD]
    mu = jnp.mean(x, axis=1, keepdims=True)
    xc = x - mu
    var = jnp.mean(xc * xc, axis=1, keepdims=True)
    y = xc * lax.rsqrt(var + 1e-6) * g_ref[...] + b_ref[...]
    for h in range(_LNB):
        o_ref[h] = y[h * _L:(h + 1) * _L, :]


def kernel(testId, assessmentItemID, KnowledgeTag, interaction, question_N,
           W_test, W_question, W_tag, W_inter, W_qN,
           W_proj, b_proj, ln_g, ln_b):
    f32 = jnp.float32
    i32 = jnp.int32
    ntest = W_test.shape[0]
    nq = W_question.shape[0]
    ntag = W_tag.shape[0]
    nqn = W_qN.shape[0]

    wq_rep = jnp.broadcast_to(W_question[:, None, :],
                              (nq, 3, _INTD)).reshape(3 * nq, _INTD)
    wi_flat = W_inter.reshape(3 * nq, _INTD)
    wtag_rep = jnp.broadcast_to(W_tag[:, None, :],
                                (ntag, nqn, _INTD)).reshape(ntag * nqn, _INTD)
    wqn_tile = jnp.broadcast_to(W_qN[None, :, :],
                                (ntag, nqn, _INTD)).reshape(ntag * nqn, _INTD)
    u = jnp.concatenate([wq_rep, wi_flat], axis=1)
    v = jnp.concatenate([wtag_rep, wqn_tile], axis=1)

    tb, tc, ta = pl.pallas_call(
        _prep_body,
        out_shape=(
            jax.ShapeDtypeStruct((3 * nq, _HD), f32),
            jax.ShapeDtypeStruct((ntag * nqn, _HD), f32),
            jax.ShapeDtypeStruct((ntest, _HD), f32),
        ),
    )(u, v, W_test, W_proj, b_proj.reshape(1, _HD))

    it, ii, ir, ig, iq = (a.reshape(_NT).astype(i32)
                          for a in (testId, assessmentItemID, interaction,
                                    KnowledgeTag, question_N))
    x = _gather_sum(it, ii, ir, ig, iq, ta, tb, tc)

    out = pl.pallas_call(
        _ln_body,
        grid=(_B // _LNB,),
        in_specs=[
            pl.BlockSpec((_LNB * _L, 2 * _HD), lambda i: (i, 0)),
            pl.BlockSpec((1, _HD), lambda i: (0, 0)),
            pl.BlockSpec((1, _HD), lambda i: (0, 0)),
        ],
        out_specs=pl.BlockSpec((_LNB, _L, _HD), lambda i: (i, 0, 0)),
        out_shape=jax.ShapeDtypeStruct((_B, _L, _HD), f32),
    )(x, ln_g.reshape(1, _HD), ln_b.reshape(1, _HD))

    return out
